# baseline (speedup 1.0000x reference)
import jax
import jax.numpy as jnp
from jax import lax
from jax.experimental import pallas as pl
from jax.experimental.pallas import tpu as pltpu
from jax.experimental.pallas import tpu_sc as plsc

B = 128
L = 1025
D = 192
SEQ = 1024
NR = 256
NM = SEQ - NR
NB = 1024
LN = 16
CH = SEQ // LN
NC, NS = 2, 16
NW = NC * NS
RPW = B // NW


def _sc_body(data_hbm, noise_hbm, out_data, out_remain, out_masked, out_revert,
             ka, va, kb, vb, hist, offs, revert_v, gidx, oidx, rows_v,
             gg, sem):
    wid = lax.axis_index("s") * NC + lax.axis_index("c")

    def row_body(r, carry):
        b = wid * RPW + r
        pltpu.sync_copy(noise_hbm.at[pl.ds(b * SEQ, SEQ)], ka)

        def init_chunk(c, carry):
            va[pl.ds(c * LN, LN)] = lax.iota(jnp.int32, LN) + c * LN
            return carry
        lax.fori_loop(0, CH, init_chunk, 0)

        for p, (ks, vs, kd, vd) in enumerate(
            ((ka, va, kb, vb), (kb, vb, ka, va), (ka, va, kb, vb))):
            shift = 10 * p

            def clear_chunk(c, carry):
                hist[pl.ds(c * LN, LN)] = jnp.zeros((LN,), jnp.int32)
                return carry
            lax.fori_loop(0, NB // LN, clear_chunk, 0)

            def hist_chunk(c, carry, ks=ks, shift=shift):
                k16 = ks[pl.ds(c * LN, LN)]
                d = (k16 >> shift) & (NB - 1)
                cnt, last = plsc.scan_count(d)
                plsc.addupdate_scatter(hist, [d], cnt, mask=last)
                return carry
            lax.fori_loop(0, CH, hist_chunk, 0)

            def scan_chunk(c, run):
                h = hist[pl.ds(c * LN, LN)]
                cs = plsc.cumsum(h)
                offs[pl.ds(c * LN, LN)] = cs - h + run
                return run + jnp.sum(h)
            lax.fori_loop(0, NB // LN, scan_chunk, jnp.int32(0))

            def perm_chunk(c, carry, ks=ks, vs=vs, kd=kd, vd=vd, shift=shift):
                k16 = ks[pl.ds(c * LN, LN)]
                v16 = vs[pl.ds(c * LN, LN)]
                d = (k16 >> shift) & (NB - 1)
                cnt, last = plsc.scan_count(d)
                base = plsc.load_gather(offs, [d])
                pos = base + cnt - 1
                plsc.store_scatter(kd, [pos], k16)
                plsc.store_scatter(vd, [pos], v16)
                plsc.addupdate_scatter(offs, [d], cnt, mask=last)
                return carry
            lax.fori_loop(0, CH, perm_chunk, 0)

        def rev_chunk(c, carry):
            v16 = vb[pl.ds(c * LN, LN)]
            plsc.store_scatter(revert_v, [v16], lax.iota(jnp.int32, LN) + c * LN)
            return carry
        lax.fori_loop(0, CH, rev_chunk, 0)

        pltpu.sync_copy(revert_v, out_revert.at[pl.ds(b * SEQ, SEQ)])
        pltpu.sync_copy(vb.at[pl.ds(0, NR)], out_remain.at[pl.ds(b * NR, NR)])
        pltpu.sync_copy(vb.at[pl.ds(NR, NM)], out_masked.at[pl.ds(b * NM, NM)])

        rowbase = b * L + 1
        dstbase = b * (NR + 1) + 1
        for j in range(2):
            def gidx_chunk(c, carry, j=j):
                v16 = vb[pl.ds(j * 128 + c * LN, LN)]
                gidx[j, pl.ds(c * LN, LN)] = v16 + rowbase
                k16 = lax.iota(jnp.int32, LN) + (j * 128 + c * LN)
                oidx[j, pl.ds(c * LN, LN)] = k16 + dstbase
                return carry
            lax.fori_loop(0, 128 // LN, gidx_chunk, 0)

        cp0 = pltpu.async_copy(data_hbm.at[gidx.at[0]],
                               rows_v.at[pl.ds(0, 128)], sem)
        cp1 = pltpu.async_copy(data_hbm.at[gidx.at[1]],
                               rows_v.at[pl.ds(128, 128)], sem)
        cp0.wait()
        cp1.wait()
        cp2 = pltpu.async_copy(rows_v.at[pl.ds(0, 128)],
                               out_data.at[oidx.at[0]], sem)
        cp3 = pltpu.async_copy(rows_v.at[pl.ds(128, 128)],
                               out_data.at[oidx.at[1]], sem)
        cp2.wait()
        cp3.wait()
        return carry

    lax.fori_loop(0, RPW, row_body, 0)

    @pl.when(wid == 0)
    def _globals():
        def gchunk(c, carry):
            b16 = lax.iota(jnp.int32, LN) + c * LN
            gidx[0, pl.ds(c * LN, LN)] = b16 * L
            oidx[0, pl.ds(c * LN, LN)] = b16 * (NR + 1)
            return carry
        lax.fori_loop(0, 128 // LN, gchunk, 0)
        cpa = pltpu.async_copy(data_hbm.at[gidx.at[0]], gg, sem)
        cpa.wait()
        cpb = pltpu.async_copy(gg, out_data.at[oidx.at[0]], sem)
        cpb.wait()


def kernel(data, noise):
    data2 = data.reshape(B * L, D)
    noise_f = lax.bitcast_convert_type(noise, jnp.int32).reshape(B * SEQ)
    mesh = plsc.VectorSubcoreMesh(core_axis_name="c", subcore_axis_name="s")
    out_type = [
        jax.ShapeDtypeStruct((B * (NR + 1), D), jnp.float32),
        jax.ShapeDtypeStruct((B * NR,), jnp.int32),
        jax.ShapeDtypeStruct((B * NM,), jnp.int32),
        jax.ShapeDtypeStruct((B * SEQ,), jnp.int32),
    ]
    scratch = [
        pltpu.VMEM((SEQ,), jnp.int32),
        pltpu.VMEM((SEQ,), jnp.int32),
        pltpu.VMEM((SEQ,), jnp.int32),
        pltpu.VMEM((SEQ,), jnp.int32),
        pltpu.VMEM((NB,), jnp.int32),
        pltpu.VMEM((NB,), jnp.int32),
        pltpu.VMEM((SEQ,), jnp.int32),
        pltpu.VMEM((2, 128), jnp.int32),
        pltpu.VMEM((2, 128), jnp.int32),
        pltpu.VMEM((NR, D), jnp.float32),
        pltpu.VMEM((128, D), jnp.float32),
        pltpu.SemaphoreType.DMA,
    ]
    run = pl.kernel(_sc_body, out_type=out_type, mesh=mesh,
                    scratch_types=scratch,
                    compiler_params=pltpu.CompilerParams(
                        needs_layout_passes=False,
                        use_tc_tiling_on_sc=False))
    od, orem, omask, orev = run(data2, noise_f)
    total_remain_data = od.reshape(B, NR + 1, D)
    remain_idx = orem.reshape(B, NR)
    masked_idx = omask.reshape(B, NM)
    revert_idx = orev.reshape(B, SEQ)
    total_remain_padding_mask = jnp.ones((B, NR + 1), jnp.float32)
    revert_padding_mask = jnp.ones((B, L), jnp.float32)
    return (total_remain_data, remain_idx, masked_idx, revert_idx,
            total_remain_padding_mask, revert_padding_mask)


# speedup vs baseline: 1.0624x; 1.0624x over previous
import jax
import jax.numpy as jnp
from jax import lax
from jax.experimental import pallas as pl
from jax.experimental.pallas import tpu as pltpu
from jax.experimental.pallas import tpu_sc as plsc

B = 128
L = 1025
D = 192
SEQ = 1024
NR = 256
NM = SEQ - NR
NB = 1024
LN = 16
CH = SEQ // LN
NC, NS = 2, 16
NW = NC * NS
RPW = B // NW


def _sc_sort_body(noise_hbm, out_remain, out_masked, out_revert,
                  ka, va, kb, vb, hist, offs, revert_v):
    wid = lax.axis_index("s") * NC + lax.axis_index("c")

    def row_body(r, carry):
        b = wid * RPW + r
        pltpu.sync_copy(noise_hbm.at[pl.ds(b * SEQ, SEQ)], ka)

        def init_chunk(c, carry):
            va[pl.ds(c * LN, LN)] = lax.iota(jnp.int32, LN) + c * LN
            return carry
        lax.fori_loop(0, CH, init_chunk, 0)

        for p, (ks, vs, kd, vd) in enumerate(
            ((ka, va, kb, vb), (kb, vb, ka, va), (ka, va, kb, vb))):
            shift = 10 * p

            def clear_chunk(c, carry):
                hist[pl.ds(c * LN, LN)] = jnp.zeros((LN,), jnp.int32)
                return carry
            lax.fori_loop(0, NB // LN, clear_chunk, 0)

            def hist_chunk(c, carry, ks=ks, shift=shift):
                k16 = ks[pl.ds(c * LN, LN)]
                d = (k16 >> shift) & (NB - 1)
                cnt, last = plsc.scan_count(d)
                plsc.addupdate_scatter(hist, [d], cnt, mask=last)
                return carry
            lax.fori_loop(0, CH, hist_chunk, 0)

            def scan_chunk(c, run):
                h = hist[pl.ds(c * LN, LN)]
                cs = plsc.cumsum(h)
                offs[pl.ds(c * LN, LN)] = cs - h + run
                return run + jnp.sum(h)
            lax.fori_loop(0, NB // LN, scan_chunk, jnp.int32(0))

            def perm_chunk(c, carry, ks=ks, vs=vs, kd=kd, vd=vd, shift=shift):
                k16 = ks[pl.ds(c * LN, LN)]
                v16 = vs[pl.ds(c * LN, LN)]
                d = (k16 >> shift) & (NB - 1)
                cnt, last = plsc.scan_count(d)
                base = plsc.load_gather(offs, [d])
                pos = base + cnt - 1
                plsc.store_scatter(kd, [pos], k16)
                plsc.store_scatter(vd, [pos], v16)
                plsc.addupdate_scatter(offs, [d], cnt, mask=last)
                return carry
            lax.fori_loop(0, CH, perm_chunk, 0)

        def rev_chunk(c, carry):
            v16 = vb[pl.ds(c * LN, LN)]
            plsc.store_scatter(revert_v, [v16], lax.iota(jnp.int32, LN) + c * LN)
            return carry
        lax.fori_loop(0, CH, rev_chunk, 0)

        pltpu.sync_copy(revert_v, out_revert.at[pl.ds(b * SEQ, SEQ)])
        pltpu.sync_copy(vb.at[pl.ds(0, NR)], out_remain.at[pl.ds(b * NR, NR)])
        pltpu.sync_copy(vb.at[pl.ds(NR, NM)], out_masked.at[pl.ds(b * NM, NM)])
        return carry

    lax.fori_loop(0, RPW, row_body, 0)


def _tc_gather_body(idx_ref, data_ref, out_ref):
    idx_row = idx_ref[0]
    jp = jnp.concatenate([jnp.zeros((1, 1), jnp.int32), idx_row + 1], axis=1)
    rows = lax.broadcasted_iota(jnp.int32, (L, NR + 1), 0)
    sel_t = (jp == rows).astype(jnp.float32)
    out_ref[0] = lax.dot_general(
        sel_t, data_ref[0], (((0,), (0,)), ((), ())),
        preferred_element_type=jnp.float32,
        precision=lax.Precision.HIGHEST)


def kernel(data, noise):
    noise_f = lax.bitcast_convert_type(noise, jnp.int32).reshape(B * SEQ)

    mesh = plsc.VectorSubcoreMesh(core_axis_name="c", subcore_axis_name="s")
    out_type = [
        jax.ShapeDtypeStruct((B * NR,), jnp.int32),
        jax.ShapeDtypeStruct((B * NM,), jnp.int32),
        jax.ShapeDtypeStruct((B * SEQ,), jnp.int32),
    ]
    scratch = [
        pltpu.VMEM((SEQ,), jnp.int32),
        pltpu.VMEM((SEQ,), jnp.int32),
        pltpu.VMEM((SEQ,), jnp.int32),
        pltpu.VMEM((SEQ,), jnp.int32),
        pltpu.VMEM((NB,), jnp.int32),
        pltpu.VMEM((NB,), jnp.int32),
        pltpu.VMEM((SEQ,), jnp.int32),
    ]
    sort_run = pl.kernel(_sc_sort_body, out_type=out_type, mesh=mesh,
                         scratch_types=scratch,
                         compiler_params=pltpu.CompilerParams(
                             needs_layout_passes=False,
                             use_tc_tiling_on_sc=False))
    orem, omask, orev = sort_run(noise_f)

    remain_idx = orem.reshape(B, NR)
    masked_idx = omask.reshape(B, NM)
    revert_idx = orev.reshape(B, SEQ)

    total_remain_data = pl.pallas_call(
        _tc_gather_body,
        grid=(B,),
        in_specs=[
            pl.BlockSpec((1, 1, NR), lambda b: (b, 0, 0)),
            pl.BlockSpec((1, L, D), lambda b: (b, 0, 0)),
        ],
        out_specs=pl.BlockSpec((1, NR + 1, D), lambda b: (b, 0, 0)),
        out_shape=jax.ShapeDtypeStruct((B, NR + 1, D), jnp.float32),
    )(remain_idx.reshape(B, 1, NR), data)

    total_remain_padding_mask = jnp.ones((B, NR + 1), jnp.float32)
    revert_padding_mask = jnp.ones((B, L), jnp.float32)
    return (total_remain_data, remain_idx, masked_idx, revert_idx,
            total_remain_padding_mask, revert_padding_mask)


# speedup vs baseline: 2.1811x; 2.0529x over previous
import jax
import jax.numpy as jnp
from jax import lax
from jax.experimental import pallas as pl
from jax.experimental.pallas import tpu as pltpu
from jax.experimental.pallas import tpu_sc as plsc

B = 128
L = 1025
D = 192
SEQ = 1024
NR = 256
NM = SEQ - NR
NB = 1024
LN = 16
CH = SEQ // LN
NC, NS = 2, 16
NW = NC * NS
RPW = B // NW


def _sc_sort_body(noise_hbm, out_remain, out_masked, out_revert,
                  ka, va, kb, vb, hist, offs, revert_v):
    wid = lax.axis_index("s") * NC + lax.axis_index("c")

    def row_body(r, carry):
        b = wid * RPW + r
        pltpu.sync_copy(noise_hbm.at[pl.ds(b * SEQ, SEQ)], ka)

        def init_chunk(c, carry):
            va[pl.ds(c * LN, LN)] = lax.iota(jnp.int32, LN) + c * LN
            return carry
        lax.fori_loop(0, CH, init_chunk, 0)

        for p, (ks, vs, kd, vd) in enumerate(
            ((ka, va, kb, vb), (kb, vb, ka, va), (ka, va, kb, vb))):
            shift = 10 * p

            def clear_chunk(c, carry):
                hist[pl.ds(c * LN, LN)] = jnp.zeros((LN,), jnp.int32)
                return carry
            lax.fori_loop(0, NB // LN, clear_chunk, 0)

            def hist_chunk(c, carry, ks=ks, shift=shift):
                k16 = ks[pl.ds(c * LN, LN)]
                d = (k16 >> shift) & (NB - 1)
                cnt, last = plsc.scan_count(d)
                plsc.addupdate_scatter(hist, [d], cnt, mask=last)
                return carry
            lax.fori_loop(0, CH, hist_chunk, 0)

            def scan_chunk(c, run):
                h = hist[pl.ds(c * LN, LN)]
                cs = plsc.cumsum(h)
                offs[pl.ds(c * LN, LN)] = cs - h + run
                return run + jnp.sum(h)
            lax.fori_loop(0, NB // LN, scan_chunk, jnp.int32(0))

            def perm_chunk(c, carry, ks=ks, vs=vs, kd=kd, vd=vd, shift=shift):
                k16 = ks[pl.ds(c * LN, LN)]
                v16 = vs[pl.ds(c * LN, LN)]
                d = (k16 >> shift) & (NB - 1)
                cnt, last = plsc.scan_count(d)
                base = plsc.load_gather(offs, [d])
                pos = base + cnt - 1
                plsc.store_scatter(kd, [pos], k16)
                plsc.store_scatter(vd, [pos], v16)
                plsc.addupdate_scatter(offs, [d], cnt, mask=last)
                return carry
            lax.fori_loop(0, CH, perm_chunk, 0)

        def rev_chunk(c, carry):
            v16 = vb[pl.ds(c * LN, LN)]
            plsc.store_scatter(revert_v, [v16], lax.iota(jnp.int32, LN) + c * LN)
            return carry
        lax.fori_loop(0, CH, rev_chunk, 0)

        pltpu.sync_copy(revert_v, out_revert.at[pl.ds(b * SEQ, SEQ)])
        pltpu.sync_copy(vb.at[pl.ds(0, NR)], out_remain.at[pl.ds(b * NR, NR)])
        pltpu.sync_copy(vb.at[pl.ds(NR, NM)], out_masked.at[pl.ds(b * NM, NM)])
        return carry

    lax.fori_loop(0, RPW, row_body, 0)


DP = 256


def _tc_transpose_body(in_ref, out_ref):
    x = in_ref[...]
    y = jnp.transpose(x, (0, 2, 1))
    pad = jnp.zeros(y.shape[:2] + (DP - D,), jnp.float32)
    out_ref[...] = jnp.concatenate([y, pad], axis=2)


def _sc_gather_body(data_hbm, remain_hbm, out_data, idx_v, gidx, rows_v, sem):
    wid = lax.axis_index("s") * NC + lax.axis_index("c")

    def row_body(r, carry):
        b = wid * RPW + r
        pltpu.sync_copy(remain_hbm.at[pl.ds(b * NR, NR)], idx_v)
        for j in range(2):
            def gidx_chunk(c, carry, j=j):
                v16 = idx_v[pl.ds(j * 128 + c * LN, LN)]
                gidx[j, pl.ds(c * LN, LN)] = ((v16 + 1) << 7) + b
                return carry
            lax.fori_loop(0, 128 // LN, gidx_chunk, 0)

        cp0 = pltpu.async_copy(data_hbm.at[gidx.at[0]],
                               rows_v.at[pl.ds(0, 128)], sem)
        cp1 = pltpu.async_copy(data_hbm.at[gidx.at[1]],
                               rows_v.at[pl.ds(128, 128)], sem)
        cp0.wait()
        cp1.wait()
        pltpu.sync_copy(rows_v, out_data.at[b])
        return carry

    lax.fori_loop(0, RPW, row_body, 0)


def kernel(data, noise):
    noise_f = lax.bitcast_convert_type(noise, jnp.int32).reshape(B * SEQ)

    mesh = plsc.VectorSubcoreMesh(core_axis_name="c", subcore_axis_name="s")
    out_type = [
        jax.ShapeDtypeStruct((B * NR,), jnp.int32),
        jax.ShapeDtypeStruct((B * NM,), jnp.int32),
        jax.ShapeDtypeStruct((B * SEQ,), jnp.int32),
    ]
    scratch = [
        pltpu.VMEM((SEQ,), jnp.int32),
        pltpu.VMEM((SEQ,), jnp.int32),
        pltpu.VMEM((SEQ,), jnp.int32),
        pltpu.VMEM((SEQ,), jnp.int32),
        pltpu.VMEM((NB,), jnp.int32),
        pltpu.VMEM((NB,), jnp.int32),
        pltpu.VMEM((SEQ,), jnp.int32),
    ]
    sort_run = pl.kernel(_sc_sort_body, out_type=out_type, mesh=mesh,
                         scratch_types=scratch,
                         compiler_params=pltpu.CompilerParams(
                             needs_layout_passes=False,
                             use_tc_tiling_on_sc=False))
    orem, omask, orev = sort_run(noise_f)

    remain_idx = orem.reshape(B, NR)
    masked_idx = omask.reshape(B, NM)
    revert_idx = orev.reshape(B, SEQ)

    data_t = jnp.transpose(data, (1, 2, 0))
    J = 25
    data_rows = pl.pallas_call(
        _tc_transpose_body,
        grid=(L // J,),
        in_specs=[pl.BlockSpec((J, D, B), lambda j: (j, 0, 0))],
        out_specs=pl.BlockSpec((J, B, DP), lambda j: (j, 0, 0)),
        out_shape=jax.ShapeDtypeStruct((L, B, DP), jnp.float32),
    )(data_t).reshape(L * B, DP)

    gather_run = pl.kernel(
        _sc_gather_body,
        out_type=jax.ShapeDtypeStruct((B, NR, DP), jnp.float32),
        mesh=mesh,
        scratch_types=[
            pltpu.VMEM((NR,), jnp.int32),
            pltpu.VMEM((2, 128), jnp.int32),
            pltpu.VMEM((NR, DP), jnp.float32),
            pltpu.SemaphoreType.DMA,
        ],
        compiler_params=pltpu.CompilerParams(
            needs_layout_passes=False,
            use_tc_tiling_on_sc=True))
    out_pad = gather_run(data_rows, orem)

    total_remain_data = jnp.concatenate(
        [data[:, :1, :], out_pad[:, :, :D]], axis=1)

    total_remain_padding_mask = jnp.ones((B, NR + 1), jnp.float32)
    revert_padding_mask = jnp.ones((B, L), jnp.float32)
    return (total_remain_data, remain_idx, masked_idx, revert_idx,
            total_remain_padding_mask, revert_padding_mask)


# speedup vs baseline: 2.2885x; 1.0493x over previous
import jax
import jax.numpy as jnp
from jax import lax
from jax.experimental import pallas as pl
from jax.experimental.pallas import tpu as pltpu
from jax.experimental.pallas import tpu_sc as plsc

B = 128
L = 1025
D = 192
SEQ = 1024
NR = 256
NM = SEQ - NR
NB = 1024
LN = 16
CH = SEQ // LN
NC, NS = 2, 16
NW = NC * NS
RPW = B // NW


def _sc_sort_body(noise_hbm, out_remain, out_masked, out_revert,
                  ka, va, kb, vb, hist, offs, revert_v):
    wid = lax.axis_index("s") * NC + lax.axis_index("c")

    def row_body(r, carry):
        b = wid * RPW + r
        pltpu.sync_copy(noise_hbm.at[pl.ds(b * SEQ, SEQ)], ka)

        def init_chunk(c, carry):
            va[pl.ds(c * LN, LN)] = lax.iota(jnp.int32, LN) + c * LN
            return carry
        lax.fori_loop(0, CH, init_chunk, 0)

        for p, (ks, vs, kd, vd) in enumerate(
            ((ka, va, kb, vb), (kb, vb, ka, va), (ka, va, kb, vb))):
            shift = 10 * p

            def clear_chunk(c, carry):
                hist[pl.ds(c * LN, LN)] = jnp.zeros((LN,), jnp.int32)
                return carry
            lax.fori_loop(0, NB // LN, clear_chunk, 0)

            def hist_chunk(c, carry, ks=ks, shift=shift):
                k16 = ks[pl.ds(c * LN, LN)]
                d = (k16 >> shift) & (NB - 1)
                cnt, last = plsc.scan_count(d)
                plsc.addupdate_scatter(hist, [d], cnt, mask=last)
                return carry
            lax.fori_loop(0, CH, hist_chunk, 0)

            def scan_chunk(c, run):
                h = hist[pl.ds(c * LN, LN)]
                cs = plsc.cumsum(h)
                offs[pl.ds(c * LN, LN)] = cs - h + run
                return run + jnp.sum(h)
            lax.fori_loop(0, NB // LN, scan_chunk, jnp.int32(0))

            def perm_chunk(c, carry, ks=ks, vs=vs, kd=kd, vd=vd, shift=shift):
                k16 = ks[pl.ds(c * LN, LN)]
                v16 = vs[pl.ds(c * LN, LN)]
                d = (k16 >> shift) & (NB - 1)
                cnt, last = plsc.scan_count(d)
                base = plsc.load_gather(offs, [d])
                pos = base + cnt - 1
                plsc.store_scatter(kd, [pos], k16)
                plsc.store_scatter(vd, [pos], v16)
                plsc.addupdate_scatter(offs, [d], cnt, mask=last)
                return carry
            lax.fori_loop(0, CH, perm_chunk, 0)

        def rev_chunk(c, carry):
            v16 = vb[pl.ds(c * LN, LN)]
            plsc.store_scatter(revert_v, [v16], lax.iota(jnp.int32, LN) + c * LN)
            return carry
        lax.fori_loop(0, CH, rev_chunk, 0)

        pltpu.sync_copy(revert_v, out_revert.at[pl.ds(b * SEQ, SEQ)])
        pltpu.sync_copy(vb.at[pl.ds(0, NR)], out_remain.at[pl.ds(b * NR, NR)])
        pltpu.sync_copy(vb.at[pl.ds(NR, NM)], out_masked.at[pl.ds(b * NM, NM)])
        return carry

    lax.fori_loop(0, RPW, row_body, 0)


DP = 256


def _tc_transpose_body(in_ref, out_ref):
    x = in_ref[...]
    y = jnp.transpose(x, (0, 2, 1))
    pad = jnp.zeros(y.shape[:2] + (DP - D,), jnp.float32)
    out_ref[...] = jnp.concatenate([y, pad], axis=2)


KP = 264


def _sc_gather_body(data_hbm, remain_hbm, out_data, idx_v, mi, mi2, rows_v,
                    sem):
    wid = lax.axis_index("s") * NC + lax.axis_index("c")

    def row_body(r, carry):
        b = wid * RPW + r
        pltpu.sync_copy(remain_hbm.at[pl.ds(b * NR, NR)], idx_v)

        iota = lax.iota(jnp.int32, LN)
        bvec = jnp.zeros((LN,), jnp.int32) + b
        mi2[...] = bvec
        zeros = jnp.zeros((LN,), jnp.int32)
        plsc.store_scatter(mi, [zeros, zeros], bvec, mask=iota == 0)

        def mi_chunk(c, carry):
            idx16 = idx_v[pl.ds(c * LN, LN)]
            g16 = ((idx16 + 1) << 7) + b
            pos = iota + (c * LN + 1)
            r16 = pos >> 7
            c16 = pos & 127
            plsc.store_scatter(mi, [r16, c16], g16, mask=pos <= 255)
            plsc.store_scatter(mi2, [jnp.maximum(pos - NR, 0)], g16,
                              mask=pos == NR)
            return carry
        lax.fori_loop(0, NR // LN, mi_chunk, 0)

        cp0 = pltpu.async_copy(data_hbm.at[mi.at[0]],
                               rows_v.at[pl.ds(0, 128)], sem)
        cp1 = pltpu.async_copy(data_hbm.at[mi.at[1]],
                               rows_v.at[pl.ds(128, 128)], sem)
        cp2 = pltpu.async_copy(data_hbm.at[mi2.at[pl.ds(0, 8)]],
                               rows_v.at[pl.ds(256, 8)], sem)
        cp0.wait()
        cp1.wait()
        cp2.wait()
        pltpu.sync_copy(rows_v, out_data.at[b])
        return carry

    lax.fori_loop(0, RPW, row_body, 0)


def _tc_transpose_back_body(in_ref, out_ref):
    x = in_ref[...]
    y = jnp.transpose(x, (1, 0, 2))
    z = jnp.transpose(y, (0, 2, 1))
    out_ref[...] = z[:, :D, :]


def kernel(data, noise):
    noise_f = lax.bitcast_convert_type(noise, jnp.int32).reshape(B * SEQ)

    mesh = plsc.VectorSubcoreMesh(core_axis_name="c", subcore_axis_name="s")
    out_type = [
        jax.ShapeDtypeStruct((B * NR,), jnp.int32),
        jax.ShapeDtypeStruct((B * NM,), jnp.int32),
        jax.ShapeDtypeStruct((B * SEQ,), jnp.int32),
    ]
    scratch = [
        pltpu.VMEM((SEQ,), jnp.int32),
        pltpu.VMEM((SEQ,), jnp.int32),
        pltpu.VMEM((SEQ,), jnp.int32),
        pltpu.VMEM((SEQ,), jnp.int32),
        pltpu.VMEM((NB,), jnp.int32),
        pltpu.VMEM((NB,), jnp.int32),
        pltpu.VMEM((SEQ,), jnp.int32),
    ]
    sort_run = pl.kernel(_sc_sort_body, out_type=out_type, mesh=mesh,
                         scratch_types=scratch,
                         compiler_params=pltpu.CompilerParams(
                             needs_layout_passes=False,
                             use_tc_tiling_on_sc=False))
    orem, omask, orev = sort_run(noise_f)

    remain_idx = orem.reshape(B, NR)
    masked_idx = omask.reshape(B, NM)
    revert_idx = orev.reshape(B, SEQ)

    data_t = jnp.transpose(data, (1, 2, 0))
    J = 25
    data_rows = pl.pallas_call(
        _tc_transpose_body,
        grid=(L // J,),
        in_specs=[pl.BlockSpec((J, D, B), lambda j: (j, 0, 0))],
        out_specs=pl.BlockSpec((J, B, DP), lambda j: (j, 0, 0)),
        out_shape=jax.ShapeDtypeStruct((L, B, DP), jnp.float32),
    )(data_t).reshape(L * B, DP)

    gather_run = pl.kernel(
        _sc_gather_body,
        out_type=jax.ShapeDtypeStruct((B, KP, DP), jnp.float32),
        mesh=mesh,
        scratch_types=[
            pltpu.VMEM((NR,), jnp.int32),
            pltpu.VMEM((2, 128), jnp.int32),
            pltpu.VMEM((LN,), jnp.int32),
            pltpu.VMEM((KP, DP), jnp.float32),
            pltpu.SemaphoreType.DMA,
        ],
        compiler_params=pltpu.CompilerParams(
            needs_layout_passes=False,
            use_tc_tiling_on_sc=True))
    out_pad = gather_run(data_rows, orem)

    out_t = pl.pallas_call(
        _tc_transpose_back_body,
        grid=(KP // 8,),
        in_specs=[pl.BlockSpec((B, 8, DP), lambda k: (0, k, 0))],
        out_specs=pl.BlockSpec((8, D, B), lambda k: (k, 0, 0)),
        out_shape=jax.ShapeDtypeStruct((NR + 1, D, B), jnp.float32),
    )(out_pad)
    total_remain_data = jnp.transpose(out_t, (2, 0, 1))

    total_remain_padding_mask = jnp.ones((B, NR + 1), jnp.float32)
    revert_padding_mask = jnp.ones((B, L), jnp.float32)
    return (total_remain_data, remain_idx, masked_idx, revert_idx,
            total_remain_padding_mask, revert_padding_mask)


# speedup vs baseline: 2.4508x; 1.0709x over previous
import jax
import jax.numpy as jnp
from jax import lax
from jax.experimental import pallas as pl
from jax.experimental.pallas import tpu as pltpu
from jax.experimental.pallas import tpu_sc as plsc

B = 128
L = 1025
D = 192
SEQ = 1024
NR = 256
NM = SEQ - NR
NB = 1024
LN = 16
CH = SEQ // LN
NC, NS = 2, 16
NW = NC * NS
RPW = B // NW


def _sc_sort_body(noise_hbm, out_remain, out_masked, out_revert,
                  ka, va, kb, vb, hist, offs, revert_v):
    wid = lax.axis_index("s") * NC + lax.axis_index("c")

    def row_body(r, carry):
        b = wid * RPW + r
        pltpu.sync_copy(noise_hbm.at[pl.ds(b * SEQ, SEQ)], ka)

        def init_chunk(c, carry):
            va[pl.ds(c * LN, LN)] = lax.iota(jnp.int32, LN) + c * LN
            return carry
        lax.fori_loop(0, CH, init_chunk, 0)

        for p, (ks, vs, kd, vd) in enumerate(
            ((ka, va, kb, vb), (kb, vb, ka, va), (ka, va, kb, vb))):
            shift = 10 * p

            def clear_chunk(c, carry):
                hist[pl.ds(c * LN, LN)] = jnp.zeros((LN,), jnp.int32)
                return carry
            lax.fori_loop(0, NB // LN, clear_chunk, 0)

            def hist_chunk(c, carry, ks=ks, shift=shift):
                k16 = ks[pl.ds(c * LN, LN)]
                d = (k16 >> shift) & (NB - 1)
                cnt, last = plsc.scan_count(d)
                plsc.addupdate_scatter(hist, [d], cnt, mask=last)
                return carry
            lax.fori_loop(0, CH, hist_chunk, 0)

            def scan_chunk(c, run):
                h = hist[pl.ds(c * LN, LN)]
                cs = plsc.cumsum(h)
                offs[pl.ds(c * LN, LN)] = cs - h + run
                return run + jnp.sum(h)
            lax.fori_loop(0, NB // LN, scan_chunk, jnp.int32(0))

            def perm_chunk(c, carry, ks=ks, vs=vs, kd=kd, vd=vd, shift=shift):
                k16 = ks[pl.ds(c * LN, LN)]
                v16 = vs[pl.ds(c * LN, LN)]
                d = (k16 >> shift) & (NB - 1)
                cnt, last = plsc.scan_count(d)
                base = plsc.load_gather(offs, [d])
                pos = base + cnt - 1
                plsc.store_scatter(kd, [pos], k16)
                plsc.store_scatter(vd, [pos], v16)
                plsc.addupdate_scatter(offs, [d], cnt, mask=last)
                return carry
            lax.fori_loop(0, CH, perm_chunk, 0)

        def rev_chunk(c, carry):
            v16 = vb[pl.ds(c * LN, LN)]
            plsc.store_scatter(revert_v, [v16], lax.iota(jnp.int32, LN) + c * LN)
            return carry
        lax.fori_loop(0, CH, rev_chunk, 0)

        pltpu.sync_copy(revert_v, out_revert.at[pl.ds(b * SEQ, SEQ)])
        pltpu.sync_copy(vb.at[pl.ds(0, NR)], out_remain.at[pl.ds(b * NR, NR)])
        pltpu.sync_copy(vb.at[pl.ds(NR, NM)], out_masked.at[pl.ds(b * NM, NM)])
        return carry

    lax.fori_loop(0, RPW, row_body, 0)


DP = 256


def _tc_transpose_body(in_ref, out_ref):
    x = in_ref[...]
    y = jnp.transpose(x, (0, 2, 1))
    pad = jnp.zeros(y.shape[:2] + (DP - D,), jnp.float32)
    out_ref[...] = jnp.concatenate([y, pad], axis=2)


KP = 264


def _sc_gather_body(data_hbm, remain_hbm, out_data, idx_v, mi, mi2, rows_v,
                    sem):
    wid = lax.axis_index("s") * NC + lax.axis_index("c")

    def row_body(r, carry):
        b = wid * RPW + r
        pltpu.sync_copy(remain_hbm.at[pl.ds(b * NR, NR)], idx_v)

        iota = lax.iota(jnp.int32, LN)
        bvec = jnp.zeros((LN,), jnp.int32) + b
        mi2[...] = bvec
        zeros = jnp.zeros((LN,), jnp.int32)
        plsc.store_scatter(mi, [zeros, zeros], bvec, mask=iota == 0)

        def mi_chunk(c, carry):
            idx16 = idx_v[pl.ds(c * LN, LN)]
            g16 = ((idx16 + 1) << 7) + b
            pos = iota + (c * LN + 1)
            r16 = pos >> 7
            c16 = pos & 127
            plsc.store_scatter(mi, [r16, c16], g16, mask=pos <= 255)
            plsc.store_scatter(mi2, [jnp.maximum(pos - NR, 0)], g16,
                              mask=pos == NR)
            return carry
        lax.fori_loop(0, NR // LN, mi_chunk, 0)

        cp0 = pltpu.async_copy(data_hbm.at[mi.at[0]],
                               rows_v.at[pl.ds(0, 128)], sem)
        cp1 = pltpu.async_copy(data_hbm.at[mi.at[1]],
                               rows_v.at[pl.ds(128, 128)], sem)
        cp2 = pltpu.async_copy(data_hbm.at[mi2.at[pl.ds(0, 8)]],
                               rows_v.at[pl.ds(256, 8)], sem)
        cp0.wait()
        cp1.wait()
        cp2.wait()
        pltpu.sync_copy(rows_v, out_data.at[b])
        return carry

    lax.fori_loop(0, RPW, row_body, 0)


def _tc_transpose_back_body(in_ref, out_ref):
    x = in_ref[...]
    y = jnp.transpose(x, (1, 0, 2))
    z = jnp.transpose(y, (0, 2, 1))
    out_ref[...] = z[:, :D, :]


def kernel(data, noise):
    noise_f = lax.bitcast_convert_type(noise, jnp.int32).reshape(B * SEQ)

    mesh = plsc.VectorSubcoreMesh(core_axis_name="c", subcore_axis_name="s")
    out_type = [
        jax.ShapeDtypeStruct((B * NR,), jnp.int32),
        jax.ShapeDtypeStruct((B * NM,), jnp.int32),
        jax.ShapeDtypeStruct((B * SEQ,), jnp.int32),
    ]
    scratch = [
        pltpu.VMEM((SEQ,), jnp.int32),
        pltpu.VMEM((SEQ,), jnp.int32),
        pltpu.VMEM((SEQ,), jnp.int32),
        pltpu.VMEM((SEQ,), jnp.int32),
        pltpu.VMEM((NB,), jnp.int32),
        pltpu.VMEM((NB,), jnp.int32),
        pltpu.VMEM((SEQ,), jnp.int32),
    ]
    sort_run = pl.kernel(_sc_sort_body, out_type=out_type, mesh=mesh,
                         scratch_types=scratch,
                         compiler_params=pltpu.CompilerParams(
                             needs_layout_passes=False,
                             use_tc_tiling_on_sc=False))
    orem, omask, orev = sort_run(noise_f)

    remain_idx = orem.reshape(B, NR)
    masked_idx = omask.reshape(B, NM)
    revert_idx = orev.reshape(B, SEQ)

    data_t = jnp.transpose(data, (1, 2, 0))
    J = 25
    data_rows = pl.pallas_call(
        _tc_transpose_body,
        grid=(L // J,),
        in_specs=[pl.BlockSpec((J, D, B), lambda j: (j, 0, 0))],
        out_specs=pl.BlockSpec((J, B, DP), lambda j: (j, 0, 0)),
        out_shape=jax.ShapeDtypeStruct((L, B, DP), jnp.float32),
    )(data_t).reshape(L * B, DP)

    gather_run = pl.kernel(
        _sc_gather_body,
        out_type=jax.ShapeDtypeStruct((B, KP, DP), jnp.float32),
        mesh=mesh,
        scratch_types=[
            pltpu.VMEM((NR,), jnp.int32),
            pltpu.VMEM((2, 128), jnp.int32),
            pltpu.VMEM((LN,), jnp.int32),
            pltpu.VMEM((KP, DP), jnp.float32),
            pltpu.SemaphoreType.DMA,
        ],
        compiler_params=pltpu.CompilerParams(
            needs_layout_passes=False,
            use_tc_tiling_on_sc=True))
    out_pad = gather_run(data_rows, orem)

    total_remain_data = out_pad[:, :NR + 1, :D]

    total_remain_padding_mask = jnp.ones((B, NR + 1), jnp.float32)
    revert_padding_mask = jnp.ones((B, L), jnp.float32)
    return (total_remain_data, remain_idx, masked_idx, revert_idx,
            total_remain_padding_mask, revert_padding_mask)


# speedup vs baseline: 2.5265x; 1.0309x over previous
import jax
import jax.numpy as jnp
from jax import lax
from jax.experimental import pallas as pl
from jax.experimental.pallas import tpu as pltpu
from jax.experimental.pallas import tpu_sc as plsc

B = 128
L = 1025
D = 192
SEQ = 1024
NR = 256
NM = SEQ - NR
NB = 1024
LN = 16
CH = SEQ // LN
NC, NS = 2, 16
NW = NC * NS
RPW = B // NW


def _sc_sort_body(noise_hbm, out_remain, out_masked, out_revert,
                  ka, va, kb, vb, hist, offs, revert_v):
    wid = lax.axis_index("s") * NC + lax.axis_index("c")

    def row_body(r, carry):
        b = wid * RPW + r
        pltpu.sync_copy(noise_hbm.at[pl.ds(b * SEQ, SEQ)], ka)

        def init_chunk(c, carry):
            va[pl.ds(c * LN, LN)] = lax.iota(jnp.int32, LN) + c * LN
            return carry
        lax.fori_loop(0, CH, init_chunk, 0)

        for p, (ks, vs, kd, vd) in enumerate(
            ((ka, va, kb, vb), (kb, vb, ka, va), (ka, va, kb, vb))):
            shift = 10 * p

            def clear_chunk(c, carry):
                hist[pl.ds(c * LN, LN)] = jnp.zeros((LN,), jnp.int32)
                return carry
            lax.fori_loop(0, NB // LN, clear_chunk, 0)

            def hist_chunk(c, carry, ks=ks, shift=shift):
                k16 = ks[pl.ds(c * LN, LN)]
                d = (k16 >> shift) & (NB - 1)
                cnt, last = plsc.scan_count(d)
                plsc.addupdate_scatter(hist, [d], cnt, mask=last)
                return carry
            lax.fori_loop(0, CH, hist_chunk, 0)

            def scan_chunk(c, run):
                h = hist[pl.ds(c * LN, LN)]
                cs = plsc.cumsum(h)
                offs[pl.ds(c * LN, LN)] = cs - h + run
                return run + jnp.sum(h)
            lax.fori_loop(0, NB // LN, scan_chunk, jnp.int32(0))

            def perm_chunk(c, carry, ks=ks, vs=vs, kd=kd, vd=vd, shift=shift):
                k16 = ks[pl.ds(c * LN, LN)]
                v16 = vs[pl.ds(c * LN, LN)]
                d = (k16 >> shift) & (NB - 1)
                cnt, last = plsc.scan_count(d)
                base = plsc.load_gather(offs, [d])
                pos = base + cnt - 1
                plsc.store_scatter(kd, [pos], k16)
                plsc.store_scatter(vd, [pos], v16)
                plsc.addupdate_scatter(offs, [d], cnt, mask=last)
                return carry
            lax.fori_loop(0, CH, perm_chunk, 0)

        def rev_chunk(c, carry):
            v16 = vb[pl.ds(c * LN, LN)]
            plsc.store_scatter(revert_v, [v16], lax.iota(jnp.int32, LN) + c * LN)
            return carry
        lax.fori_loop(0, CH, rev_chunk, 0)

        pltpu.sync_copy(revert_v, out_revert.at[pl.ds(b * SEQ, SEQ)])
        pltpu.sync_copy(vb.at[pl.ds(0, NR)], out_remain.at[pl.ds(b * NR, NR)])
        pltpu.sync_copy(vb.at[pl.ds(NR, NM)], out_masked.at[pl.ds(b * NM, NM)])
        return carry

    lax.fori_loop(0, RPW, row_body, 0)


DP = 256


def _tc_transpose_body(in_ref, out_ref):
    x = in_ref[...]
    y = jnp.transpose(x, (0, 2, 1))
    pad = jnp.zeros(y.shape[:2] + (DP - D,), jnp.float32)
    out_ref[...] = jnp.concatenate([y, pad], axis=2)


KP = 264


def _sc_gather_body(data_hbm, remain_hbm, out_data, idx_v, mi, mi2, rows_v,
                    sem):
    wid = lax.axis_index("s") * NC + lax.axis_index("c")

    def row_body(r, carry):
        b = wid * RPW + r
        pltpu.sync_copy(remain_hbm.at[pl.ds(b * NR, NR)], idx_v)

        iota = lax.iota(jnp.int32, LN)
        bvec = jnp.zeros((LN,), jnp.int32) + b
        mi2[...] = bvec
        zeros = jnp.zeros((LN,), jnp.int32)
        plsc.store_scatter(mi, [zeros, zeros], bvec, mask=iota == 0)

        def mi_chunk(c, carry):
            idx16 = idx_v[pl.ds(c * LN, LN)]
            g16 = ((idx16 + 1) << 7) + b
            pos = iota + (c * LN + 1)
            r16 = pos >> 7
            c16 = pos & 127
            plsc.store_scatter(mi, [r16, c16], g16, mask=pos <= 255)
            plsc.store_scatter(mi2, [jnp.maximum(pos - NR, 0)], g16,
                              mask=pos == NR)
            return carry
        lax.fori_loop(0, NR // LN, mi_chunk, 0)

        cp0 = pltpu.async_copy(data_hbm.at[mi.at[0]],
                               rows_v.at[pl.ds(0, 128)], sem)
        cp1 = pltpu.async_copy(data_hbm.at[mi.at[1]],
                               rows_v.at[pl.ds(128, 128)], sem)
        cp2 = pltpu.async_copy(data_hbm.at[mi2.at[pl.ds(0, 8)]],
                               rows_v.at[pl.ds(256, 8)], sem)
        cp0.wait()
        cp1.wait()
        cp2.wait()
        pltpu.sync_copy(rows_v, out_data.at[b])
        return carry

    lax.fori_loop(0, RPW, row_body, 0)


def _tc_transpose_back_body(in_ref, out_ref):
    x = in_ref[...]
    y = jnp.transpose(x, (1, 0, 2))
    z = jnp.transpose(y, (0, 2, 1))
    out_ref[...] = z[:, :D, :]


def kernel(data, noise):
    noise_f = lax.bitcast_convert_type(noise, jnp.int32).reshape(B * SEQ)

    mesh = plsc.VectorSubcoreMesh(core_axis_name="c", subcore_axis_name="s")
    out_type = [
        jax.ShapeDtypeStruct((B * NR,), jnp.int32),
        jax.ShapeDtypeStruct((B * NM,), jnp.int32),
        jax.ShapeDtypeStruct((B * SEQ,), jnp.int32),
    ]
    scratch = [
        pltpu.VMEM((SEQ,), jnp.int32),
        pltpu.VMEM((SEQ,), jnp.int32),
        pltpu.VMEM((SEQ,), jnp.int32),
        pltpu.VMEM((SEQ,), jnp.int32),
        pltpu.VMEM((NB,), jnp.int32),
        pltpu.VMEM((NB,), jnp.int32),
        pltpu.VMEM((SEQ,), jnp.int32),
    ]
    sort_run = pl.kernel(_sc_sort_body, out_type=out_type, mesh=mesh,
                         scratch_types=scratch,
                         compiler_params=pltpu.CompilerParams(
                             needs_layout_passes=False,
                             use_tc_tiling_on_sc=False))
    orem, omask, orev = sort_run(noise_f)

    remain_idx = orem.reshape(B, NR)
    masked_idx = omask.reshape(B, NM)
    revert_idx = orev.reshape(B, SEQ)

    data_t = jnp.transpose(data, (1, 2, 0))
    J = 41
    data_rows = pl.pallas_call(
        _tc_transpose_body,
        grid=(L // J,),
        in_specs=[pl.BlockSpec((J, D, B), lambda j: (j, 0, 0))],
        out_specs=pl.BlockSpec((J, B, DP), lambda j: (j, 0, 0)),
        out_shape=jax.ShapeDtypeStruct((L, B, DP), jnp.float32),
    )(data_t).reshape(L * B, DP)

    gather_run = pl.kernel(
        _sc_gather_body,
        out_type=jax.ShapeDtypeStruct((B, KP, DP), jnp.float32),
        mesh=mesh,
        scratch_types=[
            pltpu.VMEM((NR,), jnp.int32),
            pltpu.VMEM((2, 128), jnp.int32),
            pltpu.VMEM((LN,), jnp.int32),
            pltpu.VMEM((KP, DP), jnp.float32),
            pltpu.SemaphoreType.DMA,
        ],
        compiler_params=pltpu.CompilerParams(
            needs_layout_passes=False,
            use_tc_tiling_on_sc=True))
    out_pad = gather_run(data_rows, orem)

    total_remain_data = out_pad[:, :NR + 1, :D]

    total_remain_padding_mask = jnp.ones((B, NR + 1), jnp.float32)
    revert_padding_mask = jnp.ones((B, L), jnp.float32)
    return (total_remain_data, remain_idx, masked_idx, revert_idx,
            total_remain_padding_mask, revert_padding_mask)
